# baseline (speedup 1.0000x reference)
import jax
import jax.numpy as jnp
from jax.experimental import pallas as pl
from jax.experimental.pallas import tpu as pltpu

NUM_STRIPES = 4


def _copy_dma(x_ref, o_ref, sem):
    rows = x_ref.shape[0] // NUM_STRIPES
    for i in range(NUM_STRIPES):
        pltpu.make_async_copy(
            x_ref.at[pl.ds(i * rows, rows)],
            o_ref.at[pl.ds(i * rows, rows)],
            sem.at[i],
        ).start()
    for i in range(NUM_STRIPES):
        pltpu.make_async_copy(
            x_ref.at[pl.ds(i * rows, rows)],
            o_ref.at[pl.ds(i * rows, rows)],
            sem.at[i],
        ).wait()


def kernel(x):
    n, d = x.shape
    gathered = pl.pallas_call(
        _copy_dma,
        in_specs=[pl.BlockSpec(memory_space=pl.ANY)],
        out_specs=pl.BlockSpec(memory_space=pl.ANY),
        out_shape=jax.ShapeDtypeStruct((n, d), x.dtype),
        scratch_shapes=[pltpu.SemaphoreType.DMA((NUM_STRIPES,))],
    )(x)
    sizes = jnp.array([n], dtype=jnp.int32)
    return (gathered, sizes)


# speedup vs baseline: 35.1320x; 35.1320x over previous
import functools

import jax
import jax.numpy as jnp
from jax import lax
from jax.experimental import pallas as pl
from jax.experimental.pallas import tpu as pltpu
from jax.experimental.pallas import tpu_sc as plsc

NUM_WORKERS = 32
CHUNK_ROWS = 32


def _sc_copy_body(n, d, x_hbm, o_hbm, buf, insem, outsem):
    wid = lax.axis_index("s") * 2 + lax.axis_index("c")
    rows = n // NUM_WORKERS
    base = wid * rows
    nchunks = rows // CHUNK_ROWS

    def in_copy(g, slot):
        return pltpu.make_async_copy(
            x_hbm.at[pl.ds(base + g * CHUNK_ROWS, CHUNK_ROWS)],
            buf.at[slot],
            insem.at[slot],
        )

    def out_copy(g, slot):
        return pltpu.make_async_copy(
            buf.at[slot],
            o_hbm.at[pl.ds(base + g * CHUNK_ROWS, CHUNK_ROWS)],
            outsem.at[slot],
        )

    in_copy(0, 0).start()
    for g in range(nchunks):
        slot = g % 2
        if g + 1 < nchunks:
            if g >= 1:
                out_copy(g - 1, (g - 1) % 2).wait()
            in_copy(g + 1, (g + 1) % 2).start()
        in_copy(g, slot).wait()
        out_copy(g, slot).start()
    if nchunks >= 2:
        out_copy(nchunks - 2, (nchunks - 2) % 2).wait()
    out_copy(nchunks - 1, (nchunks - 1) % 2).wait()


def kernel(x):
    n, d = x.shape
    mesh = plsc.VectorSubcoreMesh(core_axis_name="c", subcore_axis_name="s")
    sc_copy = pl.kernel(
        functools.partial(_sc_copy_body, n, d),
        mesh=mesh,
        out_type=jax.ShapeDtypeStruct((n, d), x.dtype),
        scratch_types=[
            pltpu.VMEM((2, CHUNK_ROWS, d), x.dtype),
            pltpu.SemaphoreType.DMA((2,)),
            pltpu.SemaphoreType.DMA((2,)),
        ],
    )
    gathered = sc_copy(x)
    sizes = jnp.array([n], dtype=jnp.int32)
    return (gathered, sizes)
